# baseline (speedup 1.0000x reference)
import functools

import jax
import jax.numpy as jnp
import numpy as np
from jax import lax
from jax.experimental import pallas as pl
from jax.experimental.pallas import tpu as pltpu
from jax.experimental.pallas import tpu_sc as plsc

_INTERPRET = False
F32 = jnp.float32
I32 = jnp.int32

BR = 1000
EB = 128
MB = 128
MCAP = 512
SLOT = 512
NHEAD = 4
DH = 16


def _stats_body(x_ref, o_ref):
    @pl.when(pl.program_id(0) == 0)
    def _():
        o_ref[...] = jnp.zeros_like(o_ref)
    blk = x_ref[...]
    o_ref[0:1, :] += jnp.sum(blk, axis=0, keepdims=True)
    o_ref[1:2, :] += jnp.sum(blk * blk, axis=0, keepdims=True)


def _colstats(x, n_rows):
    c = x.shape[1]
    nblk = n_rows // BR
    return pl.pallas_call(
        _stats_body,
        grid=(nblk,),
        interpret=_INTERPRET,
        in_specs=[pl.BlockSpec((BR, c), lambda i: (i, 0))],
        out_specs=pl.BlockSpec((2, c), lambda i: (0, 0)),
        out_shape=jax.ShapeDtypeStruct((2, c), F32),
    )(x)


def _bn_fold(st, g, b, n_rows):
    mean = st[0] / n_rows
    var = st[1] / n_rows - mean * mean
    s = g / jnp.sqrt(var + 1e-5)
    t = b - mean * s
    return s.reshape(1, -1), t.reshape(1, -1)


def _enc_body(x_ref, pe_ref, nw_ref, nb_ref, pw_ref, pb_ref, h_ref, hs_ref):
    hx = jnp.dot(x_ref[...], nw_ref[...], preferred_element_type=F32) + nb_ref[...]
    hp = jnp.dot(pe_ref[...], pw_ref[...], preferred_element_type=F32) + pb_ref[...]
    h = jnp.concatenate([hx, hp], axis=1)
    h_ref[...] = h
    hs_ref[0] = h[:, :32]
    hs_ref[1] = h[:, 32:]


def _encoder(x, pe, nw, nb, pw, pb, n, npad):
    nblk = n // BR
    cw = nw.shape[0]
    return pl.pallas_call(
        _enc_body,
        grid=(nblk,),
        interpret=_INTERPRET,
        in_specs=[
            pl.BlockSpec((BR, cw), lambda i: (i, 0)),
            pl.BlockSpec((BR, 20), lambda i: (i, 0)),
            pl.BlockSpec(nw.shape, lambda i: (0, 0)),
            pl.BlockSpec(nb.shape, lambda i: (0, 0)),
            pl.BlockSpec(pw.shape, lambda i: (0, 0)),
            pl.BlockSpec(pb.shape, lambda i: (0, 0)),
        ],
        out_specs=[
            pl.BlockSpec((BR, 64), lambda i: (i, 0)),
            pl.BlockSpec((2, BR, 32), lambda i: (0, i, 0)),
        ],
        out_shape=[
            jax.ShapeDtypeStruct((npad, 64), F32),
            jax.ShapeDtypeStruct((2, n, 32), F32),
        ],
    )(x, pe, nw, nb, pw, pb)


def _gin_body(r_ref, sv_ref, tv_ref, agg_ref, wa_ref, ba_ref, wb_ref, bb_ref,
              wqkv_ref, bqkv_ref, r1_ref, qkv_ref):
    h = r_ref[...] * sv_ref[...] + tv_ref[...]
    agg = jnp.concatenate([agg_ref[0], agg_ref[1]], axis=1)
    gin = agg + h
    t1 = jnp.maximum(jnp.dot(gin, wa_ref[...], preferred_element_type=F32)
                     + ba_ref[...], 0.0)
    hl0 = jnp.dot(t1, wb_ref[...], preferred_element_type=F32) + bb_ref[...]
    r1_ref[...] = hl0 + h
    qkv_ref[...] = jnp.dot(h, wqkv_ref[...], preferred_element_type=F32) + bqkv_ref[...]


def _gin_qkv(r, sv, tv, agg3, wa, ba, wb, bb, wqkv, bqkv, n, npad):
    nblk = n // BR
    return pl.pallas_call(
        _gin_body,
        grid=(nblk,),
        interpret=_INTERPRET,
        in_specs=[
            pl.BlockSpec((BR, 64), lambda i: (i, 0)),
            pl.BlockSpec((1, 64), lambda i: (0, 0)),
            pl.BlockSpec((1, 64), lambda i: (0, 0)),
            pl.BlockSpec((2, BR, 32), lambda i: (0, i, 0)),
            pl.BlockSpec((64, 64), lambda i: (0, 0)),
            pl.BlockSpec((1, 64), lambda i: (0, 0)),
            pl.BlockSpec((64, 64), lambda i: (0, 0)),
            pl.BlockSpec((1, 64), lambda i: (0, 0)),
            pl.BlockSpec((64, 192), lambda i: (0, 0)),
            pl.BlockSpec((1, 192), lambda i: (0, 0)),
        ],
        out_specs=[
            pl.BlockSpec((BR, 64), lambda i: (i, 0)),
            pl.BlockSpec((BR, 192), lambda i: (i, 0)),
        ],
        out_shape=[
            jax.ShapeDtypeStruct((n, 64), F32),
            jax.ShapeDtypeStruct((npad, 192), F32),
        ],
    )(r, sv, tv, agg3, wa, ba, wb, bb, wqkv, bqkv)


def _ffn_body(r1_ref, ao_ref, r_ref, sv_ref, tv_ref,
              s1_ref, t1_ref, s2_ref, t2_ref,
              wm1_ref, bm1_ref, wm2_ref, bm2_ref, r3_ref, r3s_ref):
    r2 = ao_ref[...] + r_ref[...] * sv_ref[...] + tv_ref[...]
    o = (r1_ref[...] * s1_ref[...] + t1_ref[...]
         + r2 * s2_ref[...] + t2_ref[...])
    t = jnp.maximum(jnp.dot(o, wm1_ref[...], preferred_element_type=F32)
                    + bm1_ref[...], 0.0)
    r3 = o + jnp.dot(t, wm2_ref[...], preferred_element_type=F32) + bm2_ref[...]
    r3_ref[...] = r3
    r3s_ref[0] = r3[:, :32]
    r3s_ref[1] = r3[:, 32:]


def _bn_ffn(r1, ao, r, sv, tv, s1, t1, s2, t2, wm1, bm1, wm2, bm2, n, npad):
    nblk = n // BR
    return pl.pallas_call(
        _ffn_body,
        grid=(nblk,),
        interpret=_INTERPRET,
        in_specs=[
            pl.BlockSpec((BR, 64), lambda i: (i, 0)),
            pl.BlockSpec((BR, 64), lambda i: (i, 0)),
            pl.BlockSpec((BR, 64), lambda i: (i, 0)),
            pl.BlockSpec((1, 64), lambda i: (0, 0)),
            pl.BlockSpec((1, 64), lambda i: (0, 0)),
            pl.BlockSpec((1, 64), lambda i: (0, 0)),
            pl.BlockSpec((1, 64), lambda i: (0, 0)),
            pl.BlockSpec((1, 64), lambda i: (0, 0)),
            pl.BlockSpec((1, 64), lambda i: (0, 0)),
            pl.BlockSpec((64, 128), lambda i: (0, 0)),
            pl.BlockSpec((1, 128), lambda i: (0, 0)),
            pl.BlockSpec((128, 64), lambda i: (0, 0)),
            pl.BlockSpec((1, 64), lambda i: (0, 0)),
        ],
        out_specs=[
            pl.BlockSpec((BR, 64), lambda i: (i, 0)),
            pl.BlockSpec((2, BR, 32), lambda i: (0, i, 0)),
        ],
        out_shape=[
            jax.ShapeDtypeStruct((npad, 64), F32),
            jax.ShapeDtypeStruct((2, n, 32), F32),
        ],
    )(r1, ao, r, sv, tv, s1, t1, s2, t2, wm1, bm1, wm2, bm2)


def _head_body(r_ref, sv_ref, tv_ref, w1_ref, b1_ref, w2_ref, b2_ref,
               w3_ref, b3_ref, o_ref):
    h = r_ref[...] * sv_ref[...] + tv_ref[...]
    z = jnp.maximum(jnp.dot(h, w1_ref[...], preferred_element_type=F32)
                    + b1_ref[...], 0.0)
    z = jnp.maximum(jnp.dot(z, w2_ref[...], preferred_element_type=F32)
                    + b2_ref[...], 0.0)
    o_ref[...] = jnp.dot(z, w3_ref[...], preferred_element_type=F32) + b3_ref[...]


def _head(r, sv, tv, w1, b1, w2, b2, w3, b3, n):
    nblk = n // BR
    return pl.pallas_call(
        _head_body,
        grid=(nblk,),
        interpret=_INTERPRET,
        in_specs=[
            pl.BlockSpec((BR, 64), lambda i: (i, 0)),
            pl.BlockSpec((1, 64), lambda i: (0, 0)),
            pl.BlockSpec((1, 64), lambda i: (0, 0)),
            pl.BlockSpec((64, 32), lambda i: (0, 0)),
            pl.BlockSpec((1, 32), lambda i: (0, 0)),
            pl.BlockSpec((32, 16), lambda i: (0, 0)),
            pl.BlockSpec((1, 16), lambda i: (0, 0)),
            pl.BlockSpec((16, 1), lambda i: (0, 0)),
            pl.BlockSpec((1, 1), lambda i: (0, 0)),
        ],
        out_specs=pl.BlockSpec((BR, 1), lambda i: (i, 0)),
        out_shape=jax.ShapeDtypeStruct((n, 1), F32),
    )(r, sv, tv, w1, b1, w2, b2, w3, b3)


def _attn_body(starts_ref, counts_ref, qkv_hbm, wo_ref, bo_ref,
               ao_hbm, buf, sb, ob, lsem, wsem):
    g = pl.program_id(0)
    cnt = counts_ref[g]
    nb = (cnt + MB - 1) // MB
    slot = g * SLOT

    def load_blk(i, _):
        cp = pltpu.make_async_copy(
            qkv_hbm.at[pl.ds(slot + i * MB, MB)], buf.at[pl.ds(i * MB, MB)],
            lsem)
        cp.start()
        cp.wait()
        return 0

    lax.fori_loop(0, nb, load_blk, 0)

    colid = lax.broadcasted_iota(I32, (MB, MCAP), 1)
    kmask = colid < cnt

    def q_blk(qi, _):
        q = buf[pl.ds(qi * MB, MB), :]

        def k_blk(kb, _):
            k = buf[pl.ds(kb * MB, MB), :]
            for hh in range(NHEAD):
                s_h = lax.dot_general(
                    q[:, hh * DH:(hh + 1) * DH],
                    k[:, 64 + hh * DH:64 + (hh + 1) * DH],
                    (((1,), (1,)), ((), ())), preferred_element_type=F32)
                sb[hh, :, pl.ds(kb * MB, MB)] = s_h * 0.25
            return 0

        lax.fori_loop(0, nb, k_blk, 0)
        for hh in range(NHEAD):
            s_h = jnp.where(kmask, sb[hh], -1e9)
            m = jnp.max(s_h, axis=-1, keepdims=True)
            p_h = jnp.exp(s_h - m)
            sb[hh] = p_h / jnp.sum(p_h, axis=-1, keepdims=True)

        def av_blk(kb, acc):
            k = buf[pl.ds(kb * MB, MB), :]
            rmask = (kb * MB + lax.broadcasted_iota(I32, (MB, 1), 0)) < cnt
            parts = []
            for hh in range(NHEAD):
                p_h = sb[hh, :, pl.ds(kb * MB, MB)]
                v_h = jnp.where(rmask, k[:, 128 + hh * DH:128 + (hh + 1) * DH], 0.0)
                parts.append(jnp.dot(p_h, v_h, preferred_element_type=F32))
            return acc + jnp.concatenate(parts, axis=1)

        o = lax.fori_loop(0, nb, av_blk, jnp.zeros((MB, 64), F32))
        ob[...] = jnp.dot(o, wo_ref[...], preferred_element_type=F32) + bo_ref[...]
        wcp = pltpu.make_async_copy(
            ob, ao_hbm.at[pl.ds(slot + qi * MB, MB)], wsem)
        wcp.start()
        wcp.wait()
        return 0

    lax.fori_loop(0, nb, q_blk, 0)


def _attention(starts, counts, qkv_al, wo, bo, g_count):
    return pl.pallas_call(
        _attn_body,
        grid=(g_count,),
        interpret=_INTERPRET,
        in_specs=[
            pl.BlockSpec(memory_space=pltpu.SMEM),
            pl.BlockSpec(memory_space=pltpu.SMEM),
            pl.BlockSpec(memory_space=pl.ANY),
            pl.BlockSpec((64, 64), lambda i: (0, 0)),
            pl.BlockSpec((1, 64), lambda i: (0, 0)),
        ],
        out_specs=pl.BlockSpec(memory_space=pl.ANY),
        out_shape=jax.ShapeDtypeStruct((g_count * SLOT + 8, 64), F32),
        scratch_shapes=[
            pltpu.VMEM((MCAP, 192), F32),
            pltpu.VMEM((NHEAD, MB, MCAP), F32),
            pltpu.VMEM((MB, 64), F32),
            pltpu.SemaphoreType.DMA,
            pltpu.SemaphoreType.DMA,
        ],
    )(starts, counts, qkv_al, wo, bo)


def _stats2_body(ao_ref, r_ref, sv_ref, tv_ref, o_ref):
    @pl.when(pl.program_id(0) == 0)
    def _():
        o_ref[...] = jnp.zeros_like(o_ref)
    r2 = ao_ref[...] + r_ref[...] * sv_ref[...] + tv_ref[...]
    o_ref[0:1, :] += jnp.sum(r2, axis=0, keepdims=True)
    o_ref[1:2, :] += jnp.sum(r2 * r2, axis=0, keepdims=True)


def _colstats2(ao, r, sv, tv, n_rows):
    nblk = n_rows // BR
    return pl.pallas_call(
        _stats2_body,
        grid=(nblk,),
        interpret=_INTERPRET,
        in_specs=[
            pl.BlockSpec((BR, 64), lambda i: (i, 0)),
            pl.BlockSpec((BR, 64), lambda i: (i, 0)),
            pl.BlockSpec((1, 64), lambda i: (0, 0)),
            pl.BlockSpec((1, 64), lambda i: (0, 0)),
        ],
        out_specs=pl.BlockSpec((2, 64), lambda i: (0, 0)),
        out_shape=jax.ShapeDtypeStruct((2, 64), F32),
    )(ao, r, sv, tv)


def _sc_msg_body(n, n_chunks, table_hbm, src_hbm, dst_hbm, ea_hbm, const_hbm,
                 out_hbm, idx_v, dst_v, eat_v, rows_v, msg_v, const_v, zero_v,
                 agg_sh, sem):
    c = lax.axis_index("c")
    s = lax.axis_index("s")
    nrb = n // 1000
    nrb_my = nrb // 16 + jnp.where(s < nrb % 16, 1, 0)

    pltpu.sync_copy(const_hbm.at[c], const_v)

    def zfill(j, _):
        zero_v[j, pl.ds(0, 16)] = jnp.zeros((16,), F32)
        zero_v[j, pl.ds(16, 16)] = jnp.zeros((16,), F32)
        return 0

    lax.fori_loop(0, 200, zfill, 0)

    def zcopy(i, _):
        blk = i * 16 + s

        def zc5(j, _):
            pltpu.sync_copy(zero_v,
                            agg_sh.at[pl.ds(blk * 1000 + j * 200, 200)])
            return 0

        lax.fori_loop(0, 5, zc5, 0)
        return 0

    lax.fori_loop(0, nrb_my, zcopy, 0)
    plsc.subcore_barrier()

    w0a = const_v[pl.ds(0, 16)]
    w0b = const_v[pl.ds(16, 16)]
    w1a = const_v[pl.ds(32, 16)]
    w1b = const_v[pl.ds(48, 16)]
    eba = const_v[pl.ds(64, 16)]
    ebb = const_v[pl.ds(80, 16)]
    sca = const_v[pl.ds(96, 16)]
    scb = const_v[pl.ds(112, 16)]
    sha = const_v[pl.ds(128, 16)]
    shb = const_v[pl.ds(144, 16)]
    row_off = c * n

    n_my = n_chunks // 16 + jnp.where(s < n_chunks % 16, 1, 0)

    def chunk(i, _):
        base = (i * 16 + s) * EB
        pltpu.sync_copy(src_hbm.at[pl.ds(base, EB)], idx_v)
        pltpu.sync_copy(dst_hbm.at[pl.ds(base, EB)], dst_v)
        pltpu.sync_copy(ea_hbm.at[pl.ds(2 * base, 2 * EB)],
                        eat_v.at[pl.ds(0, 2 * EB)])

        def addoff(j, _):
            idx_v[pl.ds(j * 16, 16)] = idx_v[pl.ds(j * 16, 16)] + row_off
            return 0

        lax.fori_loop(0, EB // 16, addoff, 0)
        pltpu.async_copy(table_hbm.at[idx_v], rows_v, sem).wait()

        def edge(e, _):
            av = eat_v[pl.ds(2 * e, 16)]
            a0 = av[0]
            a1 = av[1]
            ra = rows_v[e, pl.ds(0, 16)]
            rb = rows_v[e, pl.ds(16, 16)]
            ha = ra * sca + sha
            hb = rb * scb + shb
            ma = jnp.maximum(ha + a0 * w0a + a1 * w1a + eba, 0.0)
            mb = jnp.maximum(hb + a0 * w0b + a1 * w1b + ebb, 0.0)
            msg_v[e, pl.ds(0, 16)] = ma
            msg_v[e, pl.ds(16, 16)] = mb
            return 0

        lax.fori_loop(0, EB, edge, 0)
        pltpu.sync_copy(msg_v, agg_sh.at[dst_v], add=True)
        return 0

    lax.fori_loop(0, n_my, chunk, 0)
    plsc.subcore_barrier()

    def wback(i, _):
        blk = i * 16 + s
        pltpu.sync_copy(agg_sh.at[pl.ds(blk * 1000, 1000)],
                        out_hbm.at[pl.ds(row_off + blk * 1000, 1000)])
        return 0

    lax.fori_loop(0, nrb_my, wback, 0)


def _sc_msg(table2n, src, dst, edge_attr, const, n):
    e_count = src.shape[0]
    n_chunks = e_count // EB
    mesh = plsc.VectorSubcoreMesh(core_axis_name="c", subcore_axis_name="s")
    f = pl.kernel(
        functools.partial(_sc_msg_body, n, n_chunks),
        interpret=_INTERPRET,
        compiler_params=pltpu.CompilerParams(use_tc_tiling_on_sc=False),
        out_type=jax.ShapeDtypeStruct((2 * n, 32), F32),
        mesh=mesh,
        scratch_types=[
            pltpu.VMEM((EB,), I32),
            pltpu.VMEM((EB,), I32),
            pltpu.VMEM((2 * EB + 16,), F32),
            pltpu.VMEM((EB, 32), F32),
            pltpu.VMEM((EB, 32), F32),
            pltpu.VMEM((160,), F32),
            pltpu.VMEM((200, 32), F32),
            pltpu.VMEM_SHARED((n, 32), F32),
            pltpu.SemaphoreType.DMA,
        ],
    )
    return f(table2n, src, dst, edge_attr.reshape(-1), const)


def _sc_gs_body(n_chunks, table_hbm, src_hbm, dst_hbm, out_hbm,
                srcv, dstv, rows_v, sem):
    cid = lax.axis_index("s") * 2 + lax.axis_index("c")
    n_my = n_chunks // 32 + jnp.where(cid < n_chunks % 32, 1, 0)

    def chunk(i, _):
        base = (i * 32 + cid) * EB
        pltpu.sync_copy(src_hbm.at[pl.ds(base, EB)], srcv)
        pltpu.sync_copy(dst_hbm.at[pl.ds(base, EB)], dstv)
        pltpu.async_copy(table_hbm.at[srcv], rows_v, sem).wait()
        pltpu.sync_copy(rows_v, out_hbm.at[dstv])
        return 0

    lax.fori_loop(0, n_my, chunk, 0)


def _sc_gs(table, srcmap, dstmap, out_rows, width):
    n_chunks = srcmap.shape[0] // EB
    mesh = plsc.VectorSubcoreMesh(core_axis_name="c", subcore_axis_name="s")
    f = pl.kernel(
        functools.partial(_sc_gs_body, n_chunks),
        interpret=_INTERPRET,
        compiler_params=pltpu.CompilerParams(use_tc_tiling_on_sc=False),
        out_type=jax.ShapeDtypeStruct((out_rows, width), F32),
        mesh=mesh,
        scratch_types=[
            pltpu.VMEM((EB,), I32),
            pltpu.VMEM((EB,), I32),
            pltpu.VMEM((EB, width), F32),
            pltpu.SemaphoreType.DMA,
        ],
    )
    return f(table, srcmap, dstmap)


def _pack_const(edge_w, edge_b, sv, tv):
    sv = sv.reshape(-1)
    tv = tv.reshape(-1)
    halves = []
    for c in (0, 1):
        sl = slice(c * 32, (c + 1) * 32)
        halves.append(jnp.concatenate(
            [edge_w[0, sl], edge_w[1, sl], edge_b[sl], sv[sl], tv[sl]]))
    return jnp.stack(halves)


def kernel(x, pe, edge_attr, params, edge_index, batch):
    p = params
    n = x.shape[0]
    g_count = int(p.get('_G', 0)) or 500
    npad = n + BR
    src = edge_index[0]
    dst = edge_index[1]

    pes = _colstats(pe, n)
    spe, tpe = _bn_fold(pes, p['pe_norm_g'], p['pe_norm_b'], n)
    pw_eff = spe.reshape(-1, 1) * p['pe_W']
    pb_eff = (tpe.reshape(-1) @ p['pe_W'] + p['pe_b']).reshape(1, -1)

    h0, h0s = _encoder(x, pe, p['node_W'], p['node_b'].reshape(1, -1),
                       pw_eff, pb_eff, n, npad)

    gids = jnp.arange(g_count, dtype=I32)
    starts = jnp.searchsorted(batch, gids, side='left').astype(I32)
    ends = jnp.searchsorted(batch, gids, side='right').astype(I32)
    m_cap = 4 * (n // g_count)
    counts = jnp.minimum(ends - starts, m_cap)

    lens = ((counts + MB - 1) // MB) * MB
    pstart = jnp.cumsum(lens) - lens
    total = jnp.sum(lens)
    lmax = ((n + MB * g_count) // (EB * 32) + 1) * (EB * 32)
    t = jnp.arange(lmax, dtype=I32)
    g_of_t = (jnp.searchsorted(pstart, t, side='right') - 1).astype(I32)
    pos_t = t - pstart[g_of_t]
    al_src = jnp.clip(starts[g_of_t] + pos_t, 0, npad - 1)
    dump = g_count * SLOT
    al_dst = jnp.where(t < total, g_of_t * SLOT + pos_t, dump)

    lmax2 = ((n - 1) // (EB * 32) + 1) * (EB * 32)
    t2 = jnp.arange(lmax2, dtype=I32)
    nclip = jnp.minimum(t2, n - 1)
    ua_src = jnp.clip(batch[nclip] * SLOT + nclip - starts[batch[nclip]],
                      0, g_count * SLOT - 1)
    ua_dst = jnp.where(t2 < n, t2, n)

    ones = jnp.ones((1, 64), F32)
    zeros = jnp.zeros((1, 64), F32)
    r, sv, tv = h0, ones, zeros
    table = h0s.reshape(2 * n, 32)

    for lp in p['layers']:
        const = _pack_const(p['edge_W'], p['edge_b'], sv, tv)
        agg = _sc_msg(table, src, dst, edge_attr, const, n)
        r1, qkv = _gin_qkv(r, sv, tv, agg.reshape(2, n, 32),
                           lp['Wa'], lp['ba'].reshape(1, -1),
                           lp['Wb'], lp['bb'].reshape(1, -1),
                           lp['Wqkv'], lp['bqkv'].reshape(1, -1), n, npad)
        s1, t1 = _bn_fold(_colstats(r1, n), lp['g1'], lp['b1'], n)
        qkv_al = _sc_gs(qkv, al_src, al_dst, g_count * SLOT + 8, 192)
        ao_al = _attention(starts, counts, qkv_al,
                           lp['Wo'], lp['bo'].reshape(1, -1), g_count)
        ao = _sc_gs(ao_al, ua_src, ua_dst, n + 8, 64)
        s2, t2_ = _bn_fold(_colstats2(ao, r, sv, tv, n), lp['g2'], lp['b2'], n)
        r3, r3s = _bn_ffn(r1, ao, r, sv, tv, s1, t1, s2, t2_,
                          lp['Wm1'], lp['bm1'].reshape(1, -1),
                          lp['Wm2'], lp['bm2'].reshape(1, -1), n, npad)
        s3, t3 = _bn_fold(_colstats(r3, n), lp['g3'], lp['b3'], n)
        r, sv, tv = r3, s3, t3
        table = r3s.reshape(2 * n, 32)

    return _head(r, sv, tv, p['W1'], p['b1'].reshape(1, -1),
                 p['W2'], p['b2'].reshape(1, -1),
                 p['W3'], p['b3'].reshape(1, -1), n)
